# Initial kernel scaffold; baseline (speedup 1.0000x reference)
import jax
import jax.numpy as jnp
from jax.experimental import pallas as pl


def kernel(x, edge_index, W1, b1, W2, b2, W3, b3, Wl, bl):
    raise NotImplementedError("write your pallas kernel here")


# speedup vs baseline: 8.2910x; 8.2910x over previous
import functools

import jax
import jax.numpy as jnp
from jax import lax
from jax.experimental import pallas as pl
from jax.experimental.pallas import tpu as pltpu
from jax.experimental.pallas import tpu_sc as plsc

N = 10000
E = 160000
D = 256
H = 128
NT = 32
NC = 2
NS = 16
CHUNK = 128
NCHUNK = E // CHUNK
NPAD = 10240
KPT = NPAD // (NS * CHUNK)
R = 400
GRID = N // R


def _mesh():
    return plsc.VectorSubcoreMesh(core_axis_name="c", subcore_axis_name="s",
                                  num_cores=NC, num_subcores=NS)


def _fill(ref, nrows, ncol16, value):
    def body(i, _):
        def inner(j, _):
            ref[i, pl.ds(j * 16, 16)] = jnp.full((16,), value, jnp.float32)
            return 0
        return lax.fori_loop(0, ncol16, inner, 0)
    lax.fori_loop(0, nrows, body, 0)


@functools.cache
def _build_deg_kernel():
    return functools.partial(
        pl.kernel,
        out_type=[
            jax.ShapeDtypeStruct((NPAD, H), jnp.float32),
            jax.ShapeDtypeStruct((NPAD, H), jnp.float32),
        ],
        mesh=_mesh(),
        scratch_types=[
            pltpu.VMEM((CHUNK,), jnp.int32),
            pltpu.VMEM((CHUNK,), jnp.int32),
            pltpu.VMEM((CHUNK, H), jnp.float32),
            pltpu.VMEM((CHUNK, H), jnp.float32),
            pltpu.VMEM_SHARED((NPAD, H), jnp.float32),
            pltpu.SemaphoreType.DMA,
        ],
    )(_deg_body)


def _deg_body(dst_hbm, lin_hbm, deg_a, deg_b,
              idx_v, lidx_v, ones_v, zbuf, accum, sem):
    c = lax.axis_index("c")
    s = lax.axis_index("s")
    w = s * NC + c

    _fill(ones_v, CHUNK, H // 16, 1.0)
    _fill(zbuf, CHUNK, H // 16, 0.0)
    row0 = s * (KPT * CHUNK)

    def zchunk(k, _):
        pltpu.sync_copy(lin_hbm.at[pl.ds(row0 + k * CHUNK, CHUNK)], lidx_v)
        pltpu.sync_copy(zbuf, accum.at[lidx_v])
        return 0
    lax.fori_loop(0, KPT, zchunk, 0)
    plsc.subcore_barrier()

    nfull = NCHUNK // (NC * NS)
    nrem = NCHUNK - nfull * (NC * NS)

    def body(i, _):
        j = w + i * (NC * NS)
        pltpu.sync_copy(dst_hbm.at[pl.ds(j * CHUNK, CHUNK)], idx_v)
        pltpu.sync_copy(ones_v, accum.at[idx_v], add=True)
        return 0
    lax.fori_loop(0, nfull, body, 0)

    @pl.when(w < nrem)
    def _():
        body(nfull, 0)
    plsc.subcore_barrier()

    def writeout(out_ref):
        def wchunk(k, _):
            pltpu.sync_copy(lin_hbm.at[pl.ds(row0 + k * CHUNK, CHUNK)], lidx_v)
            pltpu.async_copy(accum.at[lidx_v], zbuf, sem).wait()
            pltpu.sync_copy(zbuf, out_ref.at[pl.ds(row0 + k * CHUNK, CHUNK)])
            return 0
        lax.fori_loop(0, KPT, wchunk, 0)

    @pl.when(c == 0)
    def _():
        writeout(deg_a)

    @pl.when(c == 1)
    def _():
        writeout(deg_b)


@functools.cache
def _build_scatter_kernel():
    return functools.partial(
        pl.kernel,
        out_type=[
            jax.ShapeDtypeStruct((NPAD, H), jnp.float32),
            jax.ShapeDtypeStruct((NPAD, H), jnp.float32),
        ],
        mesh=_mesh(),
        scratch_types=[
            pltpu.VMEM((CHUNK,), jnp.int32),
            pltpu.VMEM((CHUNK,), jnp.int32),
            pltpu.VMEM((CHUNK,), jnp.int32),
            pltpu.VMEM((CHUNK, H), jnp.float32),
            pltpu.VMEM((CHUNK, H), jnp.float32),
            pltpu.VMEM_SHARED((NPAD, H), jnp.float32),
            pltpu.SemaphoreType.DMA,
        ],
    )(_scatter_body)


def _scatter_body(g_lo, g_hi, src_hbm, dst_hbm, lin_hbm, s_lo, s_hi,
                  idx_s, idx_d, lidx_v, rows, zbuf, accum, sem):
    c = lax.axis_index("c")
    s = lax.axis_index("s")

    _fill(zbuf, CHUNK, H // 16, 0.0)
    row0 = s * (KPT * CHUNK)

    def zchunk(k, _):
        pltpu.sync_copy(lin_hbm.at[pl.ds(row0 + k * CHUNK, CHUNK)], lidx_v)
        pltpu.sync_copy(zbuf, accum.at[lidx_v])
        return 0
    lax.fori_loop(0, KPT, zchunk, 0)
    plsc.subcore_barrier()

    nfull = NCHUNK // NS
    nrem = NCHUNK - nfull * NS

    def run(g_ref):
        def body(i, _):
            j = s + i * NS
            pltpu.sync_copy(src_hbm.at[pl.ds(j * CHUNK, CHUNK)], idx_s)
            pltpu.sync_copy(dst_hbm.at[pl.ds(j * CHUNK, CHUNK)], idx_d)
            pltpu.async_copy(g_ref.at[idx_s], rows, sem).wait()
            pltpu.sync_copy(rows, accum.at[idx_d], add=True)
            return 0
        lax.fori_loop(0, nfull, body, 0)

        @pl.when(s < nrem)
        def _():
            body(nfull, 0)

    @pl.when(c == 0)
    def _():
        run(g_lo)

    @pl.when(c == 1)
    def _():
        run(g_hi)

    plsc.subcore_barrier()

    def writeout(out_ref):
        def wchunk(k, _):
            pltpu.sync_copy(lin_hbm.at[pl.ds(row0 + k * CHUNK, CHUNK)], lidx_v)
            pltpu.async_copy(accum.at[lidx_v], rows, sem).wait()
            pltpu.sync_copy(rows, out_ref.at[pl.ds(row0 + k * CHUNK, CHUNK)])
            return 0
        lax.fori_loop(0, KPT, wchunk, 0)

    @pl.when(c == 0)
    def _():
        writeout(s_lo)

    @pl.when(c == 1)
    def _():
        writeout(s_hi)


def _prep_body(x_ref, w_ref, da_ref, db_ref, glo_ref, ghi_ref, dinv_ref):
    deg = da_ref[:, 0:1] + db_ref[:, 0:1] + 1.0
    dinv = jnp.where(deg > 0, lax.rsqrt(deg), 0.0)
    h = jnp.dot(x_ref[...], w_ref[...], preferred_element_type=jnp.float32)
    g = h * dinv
    glo_ref[...] = g[:, :H]
    ghi_ref[...] = g[:, H:]
    dinv_ref[...] = jnp.broadcast_to(dinv, (R, H))


def _tc_prep(x, W1, deg_a, deg_b):
    return pl.pallas_call(
        _prep_body,
        grid=(GRID,),
        in_specs=[
            pl.BlockSpec((R, D), lambda i: (i, 0)),
            pl.BlockSpec((D, D), lambda i: (0, 0)),
            pl.BlockSpec((R, H), lambda i: (i, 0)),
            pl.BlockSpec((R, H), lambda i: (i, 0)),
        ],
        out_specs=[
            pl.BlockSpec((R, H), lambda i: (i, 0)),
            pl.BlockSpec((R, H), lambda i: (i, 0)),
            pl.BlockSpec((R, H), lambda i: (i, 0)),
        ],
        out_shape=[
            jax.ShapeDtypeStruct((N, H), jnp.float32),
            jax.ShapeDtypeStruct((N, H), jnp.float32),
            jax.ShapeDtypeStruct((N, H), jnp.float32),
        ],
    )(x, W1, deg_a, deg_b)


def _mid_body(slo_ref, shi_ref, glo_ref, ghi_ref, dinv_ref, b_ref, w_ref,
              olo_ref, ohi_ref):
    dinv = dinv_ref[:, 0:1]
    pre = jnp.concatenate(
        [slo_ref[...] + glo_ref[...], shi_ref[...] + ghi_ref[...]], axis=1)
    z = jnp.maximum(pre * dinv + b_ref[...], 0.0)
    g = jnp.dot(z, w_ref[...], preferred_element_type=jnp.float32) * dinv
    olo_ref[...] = g[:, :H]
    ohi_ref[...] = g[:, H:]


def _tc_mid(s_lo, s_hi, g_lo, g_hi, dinv, b, W):
    return pl.pallas_call(
        _mid_body,
        grid=(GRID,),
        in_specs=[
            pl.BlockSpec((R, H), lambda i: (i, 0)),
            pl.BlockSpec((R, H), lambda i: (i, 0)),
            pl.BlockSpec((R, H), lambda i: (i, 0)),
            pl.BlockSpec((R, H), lambda i: (i, 0)),
            pl.BlockSpec((R, H), lambda i: (i, 0)),
            pl.BlockSpec((1, D), lambda i: (0, 0)),
            pl.BlockSpec((D, D), lambda i: (0, 0)),
        ],
        out_specs=[
            pl.BlockSpec((R, H), lambda i: (i, 0)),
            pl.BlockSpec((R, H), lambda i: (i, 0)),
        ],
        out_shape=[
            jax.ShapeDtypeStruct((N, H), jnp.float32),
            jax.ShapeDtypeStruct((N, H), jnp.float32),
        ],
    )(s_lo, s_hi, g_lo, g_hi, dinv, b.reshape(1, D), W)


def _final_body(slo_ref, shi_ref, glo_ref, ghi_ref, dinv_ref, b_ref,
                wl_ref, bl_ref, out_ref):
    dinv = dinv_ref[:, 0:1]
    pre = jnp.concatenate(
        [slo_ref[...] + glo_ref[...], shi_ref[...] + ghi_ref[...]], axis=1)
    z = jnp.maximum(pre * dinv + b_ref[...], 0.0)
    logits = jnp.dot(z, wl_ref[...], preferred_element_type=jnp.float32)
    logits = logits + bl_ref[...]
    m = jnp.max(logits, axis=1, keepdims=True)
    shifted = logits - m
    lse = jnp.log(jnp.sum(jnp.exp(shifted), axis=1, keepdims=True))
    out_ref[...] = shifted - lse


def _tc_final(s_lo, s_hi, g_lo, g_hi, dinv, b, Wl, bl):
    return pl.pallas_call(
        _final_body,
        grid=(GRID,),
        in_specs=[
            pl.BlockSpec((R, H), lambda i: (i, 0)),
            pl.BlockSpec((R, H), lambda i: (i, 0)),
            pl.BlockSpec((R, H), lambda i: (i, 0)),
            pl.BlockSpec((R, H), lambda i: (i, 0)),
            pl.BlockSpec((R, H), lambda i: (i, 0)),
            pl.BlockSpec((1, D), lambda i: (0, 0)),
            pl.BlockSpec((D, NT), lambda i: (0, 0)),
            pl.BlockSpec((1, NT), lambda i: (0, 0)),
        ],
        out_specs=pl.BlockSpec((R, NT), lambda i: (i, 0)),
        out_shape=jax.ShapeDtypeStruct((N, NT), jnp.float32),
    )(s_lo, s_hi, g_lo, g_hi, dinv, b.reshape(1, D), Wl, bl.reshape(1, NT))


@jax.jit
def kernel(x, edge_index, W1, b1, W2, b2, W3, b3, Wl, bl):
    src = edge_index[0].astype(jnp.int32)
    dst = edge_index[1].astype(jnp.int32)
    lin = jnp.arange(NPAD, dtype=jnp.int32)

    deg_kernel = _build_deg_kernel()
    scatter_kernel = _build_scatter_kernel()

    deg_a, deg_b = deg_kernel(dst, lin)
    g_lo, g_hi, dinv = _tc_prep(x, W1, deg_a, deg_b)

    s_lo, s_hi = scatter_kernel(g_lo, g_hi, src, dst, lin)
    g_lo, g_hi = _tc_mid(s_lo, s_hi, g_lo, g_hi, dinv, b1, W2)

    s_lo, s_hi = scatter_kernel(g_lo, g_hi, src, dst, lin)
    g_lo, g_hi = _tc_mid(s_lo, s_hi, g_lo, g_hi, dinv, b2, W3)

    s_lo, s_hi = scatter_kernel(g_lo, g_hi, src, dst, lin)
    return _tc_final(s_lo, s_hi, g_lo, g_hi, dinv, b3, Wl, bl)


# speedup vs baseline: 10.7623x; 1.2981x over previous
import functools

import jax
import jax.numpy as jnp
from jax import lax
from jax.experimental import pallas as pl
from jax.experimental.pallas import tpu as pltpu
from jax.experimental.pallas import tpu_sc as plsc

N = 10000
E = 160000
D = 256
H = 128
NT = 32
NC = 2
NS = 16
CHUNK = 128
NCHUNK = E // CHUNK
NPAD = 10240
KPT = NPAD // (NS * CHUNK)
R = 400
GRID = N // R


def _mesh():
    return plsc.VectorSubcoreMesh(core_axis_name="c", subcore_axis_name="s",
                                  num_cores=NC, num_subcores=NS)


def _fill(ref, nrows, ncol16, value):
    def body(i, _):
        def inner(j, _):
            ref[i, pl.ds(j * 16, 16)] = jnp.full((16,), value, jnp.float32)
            return 0
        return lax.fori_loop(0, ncol16, inner, 0)
    lax.fori_loop(0, nrows, body, 0)


@functools.cache
def _build_deg_kernel():
    return functools.partial(
        pl.kernel,
        out_type=[
            jax.ShapeDtypeStruct((NPAD, H), jnp.float32),
            jax.ShapeDtypeStruct((NPAD, H), jnp.float32),
        ],
        mesh=_mesh(),
        scratch_types=[
            pltpu.VMEM((CHUNK,), jnp.int32),
            pltpu.VMEM((CHUNK,), jnp.int32),
            pltpu.VMEM((CHUNK, H), jnp.float32),
            pltpu.VMEM((CHUNK, H), jnp.float32),
            pltpu.VMEM_SHARED((NPAD, H), jnp.float32),
            pltpu.SemaphoreType.DMA,
        ],
    )(_deg_body)


def _deg_body(dst_hbm, lin_hbm, deg_a, deg_b,
              idx_v, lidx_v, ones_v, zbuf, accum, sem):
    c = lax.axis_index("c")
    s = lax.axis_index("s")
    w = s * NC + c

    _fill(ones_v, CHUNK, H // 16, 1.0)
    _fill(zbuf, CHUNK, H // 16, 0.0)
    row0 = s * (KPT * CHUNK)

    def zchunk(k, _):
        pltpu.sync_copy(lin_hbm.at[pl.ds(row0 + k * CHUNK, CHUNK)], lidx_v)
        pltpu.sync_copy(zbuf, accum.at[lidx_v])
        return 0
    lax.fori_loop(0, KPT, zchunk, 0)
    plsc.subcore_barrier()

    nfull = NCHUNK // (NC * NS)
    nrem = NCHUNK - nfull * (NC * NS)

    def body(i, _):
        j = w + i * (NC * NS)
        pltpu.sync_copy(dst_hbm.at[pl.ds(j * CHUNK, CHUNK)], idx_v)
        pltpu.sync_copy(ones_v, accum.at[idx_v], add=True)
        return 0
    lax.fori_loop(0, nfull, body, 0)

    @pl.when(w < nrem)
    def _():
        body(nfull, 0)
    plsc.subcore_barrier()

    def writeout(out_ref):
        def wchunk(k, _):
            pltpu.sync_copy(lin_hbm.at[pl.ds(row0 + k * CHUNK, CHUNK)], lidx_v)
            pltpu.async_copy(accum.at[lidx_v], zbuf, sem).wait()
            pltpu.sync_copy(zbuf, out_ref.at[pl.ds(row0 + k * CHUNK, CHUNK)])
            return 0
        lax.fori_loop(0, KPT, wchunk, 0)

    @pl.when(c == 0)
    def _():
        writeout(deg_a)

    @pl.when(c == 1)
    def _():
        writeout(deg_b)


@functools.cache
def _build_scatter_kernel():
    return functools.partial(
        pl.kernel,
        out_type=[
            jax.ShapeDtypeStruct((NPAD, H), jnp.float32),
            jax.ShapeDtypeStruct((NPAD, H), jnp.float32),
        ],
        mesh=_mesh(),
        scratch_types=[
            pltpu.VMEM((CHUNK,), jnp.int32),
            pltpu.VMEM((CHUNK,), jnp.int32),
            pltpu.VMEM((CHUNK,), jnp.int32),
            pltpu.VMEM((CHUNK,), jnp.int32),
            pltpu.VMEM((CHUNK,), jnp.int32),
            pltpu.VMEM((CHUNK, H), jnp.float32),
            pltpu.VMEM((CHUNK, H), jnp.float32),
            pltpu.VMEM_SHARED((NPAD, H), jnp.float32),
            pltpu.SemaphoreType.DMA,
            pltpu.SemaphoreType.DMA,
        ],
    )(_scatter_body)


def _scatter_body(g_lo, g_hi, src_hbm, dst_hbm, lin_hbm, s_lo, s_hi,
                  idx_s, idx_d, idx_s2, idx_d2, lidx_v, rows, rows2,
                  accum, sem, sem2):
    c = lax.axis_index("c")
    s = lax.axis_index("s")

    _fill(rows, CHUNK, H // 16, 0.0)
    row0 = s * (KPT * CHUNK)

    def zchunk(k, _):
        pltpu.sync_copy(lin_hbm.at[pl.ds(row0 + k * CHUNK, CHUNK)], lidx_v)
        pltpu.sync_copy(rows, accum.at[lidx_v])
        return 0
    lax.fori_loop(0, KPT, zchunk, 0)
    plsc.subcore_barrier()

    nfull = NCHUNK // NS
    nrem = NCHUNK - nfull * NS

    def run(g_ref):
        def pair(i, _):
            ja = s + (2 * i) * NS
            jb = s + (2 * i + 1) * NS
            pltpu.sync_copy(src_hbm.at[pl.ds(ja * CHUNK, CHUNK)], idx_s)
            pltpu.sync_copy(dst_hbm.at[pl.ds(ja * CHUNK, CHUNK)], idx_d)
            ca = pltpu.async_copy(g_ref.at[idx_s], rows, sem)
            pltpu.sync_copy(src_hbm.at[pl.ds(jb * CHUNK, CHUNK)], idx_s2)
            pltpu.sync_copy(dst_hbm.at[pl.ds(jb * CHUNK, CHUNK)], idx_d2)
            cb = pltpu.async_copy(g_ref.at[idx_s2], rows2, sem2)
            ca.wait()
            pltpu.sync_copy(rows, accum.at[idx_d], add=True)
            cb.wait()
            pltpu.sync_copy(rows2, accum.at[idx_d2], add=True)
            return 0
        lax.fori_loop(0, nfull // 2, pair, 0)

        @pl.when(s < nrem)
        def _():
            j = s + nfull * NS
            pltpu.sync_copy(src_hbm.at[pl.ds(j * CHUNK, CHUNK)], idx_s)
            pltpu.sync_copy(dst_hbm.at[pl.ds(j * CHUNK, CHUNK)], idx_d)
            pltpu.async_copy(g_ref.at[idx_s], rows, sem).wait()
            pltpu.sync_copy(rows, accum.at[idx_d], add=True)

    @pl.when(c == 0)
    def _():
        run(g_lo)

    @pl.when(c == 1)
    def _():
        run(g_hi)

    plsc.subcore_barrier()

    def writeout(out_ref):
        def wchunk(k, _):
            pltpu.sync_copy(lin_hbm.at[pl.ds(row0 + k * CHUNK, CHUNK)], lidx_v)
            pltpu.async_copy(accum.at[lidx_v], rows, sem).wait()
            pltpu.sync_copy(rows, out_ref.at[pl.ds(row0 + k * CHUNK, CHUNK)])
            return 0
        lax.fori_loop(0, KPT, wchunk, 0)

    @pl.when(c == 0)
    def _():
        writeout(s_lo)

    @pl.when(c == 1)
    def _():
        writeout(s_hi)


def _prep_body(x_ref, w_ref, da_ref, db_ref, glo_ref, ghi_ref, dinv_ref):
    deg = da_ref[:, 0:1] + db_ref[:, 0:1] + 1.0
    dinv = jnp.where(deg > 0, lax.rsqrt(deg), 0.0)
    h = jnp.dot(x_ref[...], w_ref[...], preferred_element_type=jnp.float32)
    g = h * dinv
    glo_ref[...] = g[:, :H]
    ghi_ref[...] = g[:, H:]
    dinv_ref[...] = jnp.broadcast_to(dinv, (R, H))


def _tc_prep(x, W1, deg_a, deg_b):
    return pl.pallas_call(
        _prep_body,
        grid=(GRID,),
        in_specs=[
            pl.BlockSpec((R, D), lambda i: (i, 0)),
            pl.BlockSpec((D, D), lambda i: (0, 0)),
            pl.BlockSpec((R, H), lambda i: (i, 0)),
            pl.BlockSpec((R, H), lambda i: (i, 0)),
        ],
        out_specs=[
            pl.BlockSpec((R, H), lambda i: (i, 0)),
            pl.BlockSpec((R, H), lambda i: (i, 0)),
            pl.BlockSpec((R, H), lambda i: (i, 0)),
        ],
        out_shape=[
            jax.ShapeDtypeStruct((N, H), jnp.float32),
            jax.ShapeDtypeStruct((N, H), jnp.float32),
            jax.ShapeDtypeStruct((N, H), jnp.float32),
        ],
    )(x, W1, deg_a, deg_b)


def _mid_body(slo_ref, shi_ref, glo_ref, ghi_ref, dinv_ref, b_ref, w_ref,
              olo_ref, ohi_ref):
    dinv = dinv_ref[:, 0:1]
    pre = jnp.concatenate(
        [slo_ref[...] + glo_ref[...], shi_ref[...] + ghi_ref[...]], axis=1)
    z = jnp.maximum(pre * dinv + b_ref[...], 0.0)
    g = jnp.dot(z, w_ref[...], preferred_element_type=jnp.float32) * dinv
    olo_ref[...] = g[:, :H]
    ohi_ref[...] = g[:, H:]


def _tc_mid(s_lo, s_hi, g_lo, g_hi, dinv, b, W):
    return pl.pallas_call(
        _mid_body,
        grid=(GRID,),
        in_specs=[
            pl.BlockSpec((R, H), lambda i: (i, 0)),
            pl.BlockSpec((R, H), lambda i: (i, 0)),
            pl.BlockSpec((R, H), lambda i: (i, 0)),
            pl.BlockSpec((R, H), lambda i: (i, 0)),
            pl.BlockSpec((R, H), lambda i: (i, 0)),
            pl.BlockSpec((1, D), lambda i: (0, 0)),
            pl.BlockSpec((D, D), lambda i: (0, 0)),
        ],
        out_specs=[
            pl.BlockSpec((R, H), lambda i: (i, 0)),
            pl.BlockSpec((R, H), lambda i: (i, 0)),
        ],
        out_shape=[
            jax.ShapeDtypeStruct((N, H), jnp.float32),
            jax.ShapeDtypeStruct((N, H), jnp.float32),
        ],
    )(s_lo, s_hi, g_lo, g_hi, dinv, b.reshape(1, D), W)


def _final_body(slo_ref, shi_ref, glo_ref, ghi_ref, dinv_ref, b_ref,
                wl_ref, bl_ref, out_ref):
    dinv = dinv_ref[:, 0:1]
    pre = jnp.concatenate(
        [slo_ref[...] + glo_ref[...], shi_ref[...] + ghi_ref[...]], axis=1)
    z = jnp.maximum(pre * dinv + b_ref[...], 0.0)
    logits = jnp.dot(z, wl_ref[...], preferred_element_type=jnp.float32)
    logits = logits + bl_ref[...]
    m = jnp.max(logits, axis=1, keepdims=True)
    shifted = logits - m
    lse = jnp.log(jnp.sum(jnp.exp(shifted), axis=1, keepdims=True))
    out_ref[...] = shifted - lse


def _tc_final(s_lo, s_hi, g_lo, g_hi, dinv, b, Wl, bl):
    return pl.pallas_call(
        _final_body,
        grid=(GRID,),
        in_specs=[
            pl.BlockSpec((R, H), lambda i: (i, 0)),
            pl.BlockSpec((R, H), lambda i: (i, 0)),
            pl.BlockSpec((R, H), lambda i: (i, 0)),
            pl.BlockSpec((R, H), lambda i: (i, 0)),
            pl.BlockSpec((R, H), lambda i: (i, 0)),
            pl.BlockSpec((1, D), lambda i: (0, 0)),
            pl.BlockSpec((D, NT), lambda i: (0, 0)),
            pl.BlockSpec((1, NT), lambda i: (0, 0)),
        ],
        out_specs=pl.BlockSpec((R, NT), lambda i: (i, 0)),
        out_shape=jax.ShapeDtypeStruct((N, NT), jnp.float32),
    )(s_lo, s_hi, g_lo, g_hi, dinv, b.reshape(1, D), Wl, bl.reshape(1, NT))


@jax.jit
def kernel(x, edge_index, W1, b1, W2, b2, W3, b3, Wl, bl):
    src = edge_index[0].astype(jnp.int32)
    dst = edge_index[1].astype(jnp.int32)
    lin = jnp.arange(NPAD, dtype=jnp.int32)

    deg_kernel = _build_deg_kernel()
    scatter_kernel = _build_scatter_kernel()

    deg_a, deg_b = deg_kernel(dst, lin)
    g_lo, g_hi, dinv = _tc_prep(x, W1, deg_a, deg_b)

    s_lo, s_hi = scatter_kernel(g_lo, g_hi, src, dst, lin)
    g_lo, g_hi = _tc_mid(s_lo, s_hi, g_lo, g_hi, dinv, b1, W2)

    s_lo, s_hi = scatter_kernel(g_lo, g_hi, src, dst, lin)
    g_lo, g_hi = _tc_mid(s_lo, s_hi, g_lo, g_hi, dinv, b2, W3)

    s_lo, s_hi = scatter_kernel(g_lo, g_hi, src, dst, lin)
    return _tc_final(s_lo, s_hi, g_lo, g_hi, dinv, b3, Wl, bl)
